# baseline (speedup 1.0000x reference)
import functools
import math

import jax
import jax.numpy as jnp
from jax import lax
from jax.experimental import pallas as pl
from jax.experimental.pallas import tpu as pltpu
from jax.experimental.pallas import tpu_sc as plsc


_TT = 512


def _score_body(x_ref, w_ref, o_ref):
    o_ref[...] = jnp.dot(x_ref[...], w_ref[...],
                         preferred_element_type=jnp.float32)


def _scores(x2, w2):
    nbt = x2.shape[0] // _TT
    d = x2.shape[1]
    return pl.pallas_call(
        _score_body,
        grid=(nbt,),
        in_specs=[
            pl.BlockSpec((_TT, d), lambda i: (i, 0)),
            pl.BlockSpec((d, 1), lambda i: (0, 0)),
        ],
        out_specs=pl.BlockSpec((_TT, 1), lambda i: (i, 0)),
        out_shape=jax.ShapeDtypeStruct((x2.shape[0], 1), jnp.float32),
    )(x2, w2)


_RC = 512


def _sortable(v):
    u = lax.bitcast_convert_type(v, jnp.int32)
    return u ^ (lax.shift_right_arithmetic(u, 31) & jnp.int32(0x7FFFFFFF))


def _rank_body(scol_ref, srow_ref, rank_ref):
    b_sz, t = srow_ref.shape
    n = t // _RC
    ones_col = jnp.ones((_RC, 1), jnp.float32)
    jl_diag = (lax.broadcasted_iota(jnp.int32, (_RC, _RC), 1)
               < lax.broadcasted_iota(jnp.int32, (_RC, _RC), 0))
    dn_col = (((0,), (0,)), ((), ()))
    for b in range(b_sz):
        ks_col = _sortable(scol_ref[b * t:(b + 1) * t, :])
        ks_row = _sortable(srow_ref[b:b + 1, :])
        acc = [jnp.full((_RC, 1), float(a * _RC), jnp.float32)
               for a in range(n)]
        for a in range(n):
            ks_a = ks_col[a * _RC:(a + 1) * _RC, :]
            ksd = ks_row[:, a * _RC:(a + 1) * _RC]
            diag = ((ksd > ks_a) | ((ksd == ks_a) & jl_diag)
                    ).astype(jnp.float32)
            acc[a] = acc[a] + jnp.dot(diag, ones_col,
                                      preferred_element_type=jnp.float32)
            for c in range(a + 1, n):
                m = (ks_row[:, c * _RC:(c + 1) * _RC] > ks_a
                     ).astype(jnp.float32)
                acc[a] = acc[a] + jnp.dot(m, ones_col,
                                          preferred_element_type=jnp.float32)
                acc[c] = acc[c] - lax.dot_general(
                    m, ones_col, dn_col,
                    preferred_element_type=jnp.float32)
        rank = jnp.concatenate(acc, axis=0)
        rank_ref[b * t:(b + 1) * t, :] = rank.astype(jnp.int32)


def _ranks(scol, srow):
    b, t = srow.shape
    return pl.pallas_call(
        _rank_body,
        out_shape=jax.ShapeDtypeStruct((b * t, 1), jnp.int32),
    )(scol, srow)


_CH = 16


_KC = 512


def _tc_invert_body(rank_ref, idx_ref, gidx_ref):
    b_sz, k = idx_ref.shape
    t = rank_ref.shape[0] // b_sz
    iota_row = lax.broadcasted_iota(jnp.int32, (1, t), 1).astype(jnp.float32)
    for b in range(b_sz):
        rank = rank_ref[b * t:(b + 1) * t, :]
        for kc in range(k // _KC):
            rvals = lax.broadcasted_iota(jnp.int32, (t, _KC), 1) + kc * _KC
            hit = (rank == rvals).astype(jnp.float32)
            contrib = jnp.dot(iota_row, hit,
                              preferred_element_type=jnp.float32)
            ci = contrib.astype(jnp.int32)[0, :]
            idx_ref[b, kc * _KC:(kc + 1) * _KC] = ci
            gidx_ref[b, kc * _KC:(kc + 1) * _KC] = ci + b * t


def _tc_invert(rank, b, t, k):
    return pl.pallas_call(
        _tc_invert_body,
        out_shape=(
            jax.ShapeDtypeStruct((b, k), jnp.int32),
            jax.ShapeDtypeStruct((b, k), jnp.int32),
        ),
    )(rank)


def _make_sc_invert(b_sz, t, k):
    n_rows = b_sz * k
    info = plsc.get_sparse_core_info()
    nw = info.num_cores * info.num_subcores
    nc = info.num_cores
    rpw = n_rows // nw
    wpb = k // rpw
    mesh = plsc.VectorSubcoreMesh(core_axis_name="c", subcore_axis_name="s")

    @functools.partial(
        pl.kernel,
        mesh=mesh,
        compiler_params=pltpu.CompilerParams(needs_layout_passes=False),
        out_type=(
            jax.ShapeDtypeStruct((n_rows,), jnp.int32),
            jax.ShapeDtypeStruct((n_rows,), jnp.int32),
        ),
        scratch_types=[
            pltpu.VMEM((t,), jnp.int32),
            pltpu.VMEM((rpw,), jnp.int32),
            pltpu.VMEM((rpw,), jnp.int32),
        ],
    )
    def invert_k(rank_hbm, idxout_hbm, gidxout_hbm, rank_v, idxl_v, idxg_v):
        wid = lax.axis_index("s") * nc + lax.axis_index("c")
        b = wid // wpb
        slot_lo = (wid % wpb) * rpw
        base = wid * rpw
        pltpu.sync_copy(rank_hbm.at[pl.ds(b * t, t)], rank_v)
        iota16 = lax.broadcasted_iota(jnp.int32, (16,), 0)
        gbase = b * t

        def inv_body(i, _):
            r = rank_v[pl.ds(i * 16, 16)]
            rloc = r - slot_lo
            msk = (rloc >= 0) & (rloc < rpw)
            vloc = iota16 + i * 16
            plsc.store_scatter(idxl_v, [rloc], vloc, mask=msk)
            plsc.store_scatter(idxg_v, [rloc], vloc + gbase, mask=msk)
            return 0

        lax.fori_loop(0, t // 16, inv_body, 0)
        pltpu.sync_copy(idxl_v, idxout_hbm.at[pl.ds(base, rpw)])
        pltpu.sync_copy(idxg_v, gidxout_hbm.at[pl.ds(base, rpw)])

    return invert_k


def _make_sc_gather(n_rows, d):
    info = plsc.get_sparse_core_info()
    nw = info.num_cores * info.num_subcores
    nc = info.num_cores
    b_per_w = n_rows // nw
    n_ch = b_per_w // _CH
    mesh = plsc.VectorSubcoreMesh(core_axis_name="c", subcore_axis_name="s")

    @functools.partial(
        pl.kernel,
        mesh=mesh,
        out_type=jax.ShapeDtypeStruct((n_rows, d), jnp.float32),
        scratch_types=[
            pltpu.VMEM((_CH,), jnp.int32),
            pltpu.VMEM((_CH,), jnp.int32),
            pltpu.VMEM((_CH, d), jnp.float32),
            pltpu.VMEM((_CH, d), jnp.float32),
            pltpu.SemaphoreType.DMA,
            pltpu.SemaphoreType.DMA,
            pltpu.SemaphoreType.DMA,
            pltpu.SemaphoreType.DMA,
        ],
    )
    def gather_k(table_hbm, idx_hbm, out_hbm,
                 idx_v0, idx_v1, rows_v0, rows_v1,
                 sem_g0, sem_g1, sem_o0, sem_o1):
        wid = lax.axis_index("s") * nc + lax.axis_index("c")
        base = wid * b_per_w
        idx_v = [idx_v0, idx_v1]
        rows_v = [rows_v0, rows_v1]
        sem_g = [sem_g0, sem_g1]
        sem_o = [sem_o0, sem_o1]
        g = [None] * n_ch
        w = [None] * n_ch
        for c in range(n_ch):
            p = c % 2
            if c == 0:
                pltpu.sync_copy(idx_hbm.at[pl.ds(base, _CH)], idx_v[0])
                g[0] = pltpu.async_copy(table_hbm.at[idx_v[0]], rows_v[0],
                                        sem_g[0])
            if c + 1 < n_ch:
                pn = (c + 1) % 2
                pltpu.sync_copy(
                    idx_hbm.at[pl.ds(base + (c + 1) * _CH, _CH)], idx_v[pn])
                if c >= 1:
                    w[c - 1].wait()
                g[c + 1] = pltpu.async_copy(table_hbm.at[idx_v[pn]],
                                            rows_v[pn], sem_g[pn])
            g[c].wait()
            w[c] = pltpu.async_copy(
                rows_v[p], out_hbm.at[pl.ds(base + c * _CH, _CH)], sem_o[p])
        if n_ch >= 2:
            w[n_ch - 2].wait()
        w[n_ch - 1].wait()

    return gather_k


def kernel(x, W):
    b, t, d = x.shape
    k = max(1, math.ceil(0.5 * t))

    x2 = x.reshape(b * t, d)
    scol = _scores(x2, W.reshape(d, 1))
    srow = scol.reshape(b, t)
    rank = _ranks(scol, srow)

    indices, gidx = _tc_invert(rank, b, t, k)
    gather_fn = _make_sc_gather(b * k, d)
    selected = gather_fn(x2, gidx.reshape(b * k))
    return selected.reshape(b, k, d), indices, srow


# speedup vs baseline: 1.0816x; 1.0816x over previous
import functools
import math

import jax
import jax.numpy as jnp
from jax import lax
from jax.experimental import pallas as pl
from jax.experimental.pallas import tpu as pltpu
from jax.experimental.pallas import tpu_sc as plsc


_TT = 512


def _score_body(x_ref, w_ref, o_ref):
    o_ref[0] = lax.dot_general(
        w_ref[...], x_ref[...], (((1,), (1,)), ((), ())),
        preferred_element_type=jnp.float32)


def _scores(x2, w2r):
    nbt = x2.shape[0] // _TT
    d = x2.shape[1]
    return pl.pallas_call(
        _score_body,
        grid=(nbt,),
        in_specs=[
            pl.BlockSpec((_TT, d), lambda i: (i, 0)),
            pl.BlockSpec((1, d), lambda i: (0, 0)),
        ],
        out_specs=pl.BlockSpec((1, 1, _TT), lambda i: (i, 0, 0)),
        out_shape=jax.ShapeDtypeStruct((nbt, 1, _TT), jnp.float32),
    )(x2, w2r)


_RC = 512
_KC = 512


def _sortable(v):
    u = lax.bitcast_convert_type(v, jnp.int32)
    return u ^ (lax.shift_right_arithmetic(u, 31) & jnp.int32(0x7FFFFFFF))


def _topk_body(scol_ref, srow_ref, idx_ref, gidx_ref):
    b_sz, t = srow_ref.shape
    k = idx_ref.shape[1]
    ones_col = jnp.ones((t, 1), jnp.float32)
    iota_row = lax.broadcasted_iota(jnp.int32, (1, t), 1).astype(jnp.float32)
    jl_diag = (lax.broadcasted_iota(jnp.int32, (_RC, _RC), 1)
               < lax.broadcasted_iota(jnp.int32, (_RC, _RC), 0))
    for b in range(b_sz):
        ks_col = _sortable(scol_ref[b * t:(b + 1) * t, :])
        ks_row = _sortable(srow_ref[b:b + 1, :])
        acc_blocks = []
        for ic in range(t // _RC):
            lo, hi = ic * _RC, (ic + 1) * _RC
            ks_i = ks_col[lo:hi, :]
            parts = []
            if lo > 0:
                parts.append((ks_row[:, :lo] >= ks_i).astype(jnp.float32))
            ksd = ks_row[:, lo:hi]
            diag = (ksd > ks_i) | ((ksd == ks_i) & jl_diag)
            parts.append(diag.astype(jnp.float32))
            if hi < t:
                parts.append((ks_row[:, hi:] > ks_i).astype(jnp.float32))
            beats = jnp.concatenate(parts, axis=1)
            acc_blocks.append(jnp.dot(beats, ones_col,
                                      preferred_element_type=jnp.float32))
        rank = jnp.concatenate(acc_blocks, axis=0)
        for kc in range(k // _KC):
            rvals = (lax.broadcasted_iota(jnp.int32, (t, _KC), 1)
                     + kc * _KC).astype(jnp.float32)
            hit = (rank == rvals).astype(jnp.float32)
            contrib = jnp.dot(iota_row, hit,
                              preferred_element_type=jnp.float32)
            ci = contrib.astype(jnp.int32)[0, :]
            idx_ref[b, kc * _KC:(kc + 1) * _KC] = ci
            gidx_ref[b, kc * _KC:(kc + 1) * _KC] = ci + b * t


def _topk(scol, srow, k):
    b, t = srow.shape
    return pl.pallas_call(
        _topk_body,
        out_shape=(
            jax.ShapeDtypeStruct((b, k), jnp.int32),
            jax.ShapeDtypeStruct((b, k), jnp.int32),
        ),
    )(scol, srow)


_CH = 16


def _make_sc_gather(n_rows, d):
    info = plsc.get_sparse_core_info()
    nw = info.num_cores * info.num_subcores
    nc = info.num_cores
    b_per_w = n_rows // nw
    n_ch = b_per_w // _CH
    mesh = plsc.VectorSubcoreMesh(core_axis_name="c", subcore_axis_name="s")

    @functools.partial(
        pl.kernel,
        mesh=mesh,
        out_type=jax.ShapeDtypeStruct((n_rows, d), jnp.float32),
        scratch_types=[
            pltpu.VMEM((b_per_w,), jnp.int32),
            pltpu.VMEM((_CH,), jnp.int32),
            pltpu.VMEM((_CH,), jnp.int32),
            pltpu.VMEM((_CH, d), jnp.float32),
            pltpu.VMEM((_CH, d), jnp.float32),
            pltpu.SemaphoreType.DMA,
            pltpu.SemaphoreType.DMA,
            pltpu.SemaphoreType.DMA,
            pltpu.SemaphoreType.DMA,
        ],
    )
    def gather_k(table_hbm, idx_hbm, out_hbm,
                 idx_all, i16_0, i16_1, rows_v0, rows_v1,
                 sem_g0, sem_g1, sem_o0, sem_o1):
        wid = lax.axis_index("s") * nc + lax.axis_index("c")
        base = wid * b_per_w
        pltpu.sync_copy(idx_hbm.at[pl.ds(base, b_per_w)], idx_all)
        idx16 = [i16_0, i16_1]
        rows_v = [rows_v0, rows_v1]
        sem_g = [sem_g0, sem_g1]
        sem_o = [sem_o0, sem_o1]
        g = [None] * n_ch
        w = [None] * n_ch
        for c in range(n_ch):
            p = c % 2
            if c == 0:
                idx16[0][...] = idx_all[pl.ds(0, _CH)]
                g[0] = pltpu.async_copy(table_hbm.at[idx16[0]], rows_v[0],
                                        sem_g[0])
            if c + 1 < n_ch:
                pn = (c + 1) % 2
                idx16[pn][...] = idx_all[pl.ds((c + 1) * _CH, _CH)]
                if c >= 1:
                    w[c - 1].wait()
                g[c + 1] = pltpu.async_copy(table_hbm.at[idx16[pn]],
                                            rows_v[pn], sem_g[pn])
            g[c].wait()
            w[c] = pltpu.async_copy(
                rows_v[p], out_hbm.at[pl.ds(base + c * _CH, _CH)], sem_o[p])
        if n_ch >= 2:
            w[n_ch - 2].wait()
        w[n_ch - 1].wait()

    return gather_k


def kernel(x, W):
    b, t, d = x.shape
    k = max(1, math.ceil(0.5 * t))

    x2 = x.reshape(b * t, d)
    srow = _scores(x2, W.reshape(1, d)).reshape(b, t)
    scol = srow.reshape(b * t, 1)
    indices, gidx = _topk(scol, srow, k)

    gather_fn = _make_sc_gather(b * k, d)
    selected = gather_fn(x2, gidx.reshape(b * k))
    return selected.reshape(b, k, d), indices, srow


# speedup vs baseline: 1.2096x; 1.1183x over previous
import functools
import math

import jax
import jax.numpy as jnp
from jax import lax
from jax.experimental import pallas as pl
from jax.experimental.pallas import tpu as pltpu
from jax.experimental.pallas import tpu_sc as plsc


_TT = 2048


def _score_body(x_ref, w_ref, o_ref):
    o_ref[0] = lax.dot_general(
        w_ref[...], x_ref[...], (((1,), (1,)), ((), ())),
        preferred_element_type=jnp.float32)


def _scores(x2, w2r):
    nbt = x2.shape[0] // _TT
    d = x2.shape[1]
    return pl.pallas_call(
        _score_body,
        grid=(nbt,),
        in_specs=[
            pl.BlockSpec((_TT, d), lambda i: (i, 0)),
            pl.BlockSpec((1, d), lambda i: (0, 0)),
        ],
        out_specs=pl.BlockSpec((1, 1, _TT), lambda i: (i, 0, 0)),
        out_shape=jax.ShapeDtypeStruct((nbt, 1, _TT), jnp.float32),
    )(x2, w2r)


_RC = 512
_LB = 64


def _sortable(v):
    u = lax.bitcast_convert_type(v, jnp.int32)
    return u ^ (lax.shift_right_arithmetic(u, 31) & jnp.int32(0x7FFFFFFF))


def _topk_body(srow_ref, idx_ref, gidx_ref):
    b_sz, t = srow_ref.shape
    kh = idx_ref.shape[1]
    ones_col = jnp.ones((t, 1), jnp.float32)
    iota_col = lax.broadcasted_iota(jnp.int32, (t, 1), 0).astype(jnp.float32)
    hi_iota = lax.broadcasted_iota(jnp.int32, (kh, 1), 0)
    lo_iota = lax.broadcasted_iota(jnp.int32, (1, _LB), 1)
    jl_diag = (lax.broadcasted_iota(jnp.int32, (_RC, _RC), 1)
               < lax.broadcasted_iota(jnp.int32, (_RC, _RC), 0))
    for b in range(b_sz):
        ks_row = _sortable(srow_ref[b:b + 1, :])
        ks_col = jnp.transpose(ks_row, (1, 0))
        acc_blocks = []
        for ic in range(t // _RC):
            lo, hi = ic * _RC, (ic + 1) * _RC
            ks_i = ks_col[lo:hi, :]
            a = jnp.zeros((_RC, 1), jnp.float32)
            if lo > 0:
                ge = (ks_row[:, :lo] >= ks_i).astype(jnp.float32)
                a = a + jnp.dot(ge, ones_col[:lo, :],
                                preferred_element_type=jnp.float32)
            ksd = ks_row[:, lo:hi]
            diag = ((ksd > ks_i) | ((ksd == ks_i) & jl_diag)
                    ).astype(jnp.float32)
            a = a + jnp.dot(diag, ones_col[:_RC, :],
                            preferred_element_type=jnp.float32)
            if hi < t:
                gt = (ks_row[:, hi:] > ks_i).astype(jnp.float32)
                a = a + jnp.dot(gt, ones_col[:t - hi, :],
                                preferred_element_type=jnp.float32)
            acc_blocks.append(a)
        rank = jnp.concatenate(acc_blocks, axis=0).astype(jnp.int32)
        rank_row = jnp.transpose(rank, (1, 0))
        h_t = (hi_iota == lax.shift_right_logical(rank_row, 6)
               ).astype(jnp.float32)
        l_m = ((rank & jnp.int32(_LB - 1)) == lo_iota
               ).astype(jnp.float32)
        il = l_m * iota_col
        a_idx = jnp.dot(h_t, il,
                        preferred_element_type=jnp.float32)
        ai = a_idx.astype(jnp.int32)
        idx_ref[b] = ai
        gidx_ref[b] = ai + b * t


def _topk(srow, k):
    b, t = srow.shape
    kh = k // _LB
    return pl.pallas_call(
        _topk_body,
        out_shape=(
            jax.ShapeDtypeStruct((b, kh, _LB), jnp.int32),
            jax.ShapeDtypeStruct((b, kh, _LB), jnp.int32),
        ),
    )(srow)


_CH = 16


def _make_sc_gather(n_rows, d):
    info = plsc.get_sparse_core_info()
    nw = info.num_cores * info.num_subcores
    nc = info.num_cores
    b_per_w = n_rows // nw
    n_ch = b_per_w // _CH
    mesh = plsc.VectorSubcoreMesh(core_axis_name="c", subcore_axis_name="s")

    @functools.partial(
        pl.kernel,
        mesh=mesh,
        out_type=jax.ShapeDtypeStruct((n_rows, d), jnp.float32),
        scratch_types=[
            pltpu.VMEM((b_per_w,), jnp.int32),
            pltpu.VMEM((_CH,), jnp.int32),
            pltpu.VMEM((_CH,), jnp.int32),
            pltpu.VMEM((_CH, d), jnp.float32),
            pltpu.VMEM((_CH, d), jnp.float32),
            pltpu.SemaphoreType.DMA,
            pltpu.SemaphoreType.DMA,
            pltpu.SemaphoreType.DMA,
            pltpu.SemaphoreType.DMA,
        ],
    )
    def gather_k(table_hbm, idx_hbm, out_hbm,
                 idx_all, i16_0, i16_1, rows_v0, rows_v1,
                 sem_g0, sem_g1, sem_o0, sem_o1):
        wid = lax.axis_index("s") * nc + lax.axis_index("c")
        base = wid * b_per_w
        pltpu.sync_copy(idx_hbm.at[pl.ds(base, b_per_w)], idx_all)
        idx16 = [i16_0, i16_1]
        rows_v = [rows_v0, rows_v1]
        sem_g = [sem_g0, sem_g1]
        sem_o = [sem_o0, sem_o1]
        g = [None] * n_ch
        w = [None] * n_ch
        for c in range(n_ch):
            p = c % 2
            if c == 0:
                idx16[0][...] = idx_all[pl.ds(0, _CH)]
                g[0] = pltpu.async_copy(table_hbm.at[idx16[0]], rows_v[0],
                                        sem_g[0])
            if c + 1 < n_ch:
                pn = (c + 1) % 2
                idx16[pn][...] = idx_all[pl.ds((c + 1) * _CH, _CH)]
                if c >= 1:
                    w[c - 1].wait()
                g[c + 1] = pltpu.async_copy(table_hbm.at[idx16[pn]],
                                            rows_v[pn], sem_g[pn])
            g[c].wait()
            w[c] = pltpu.async_copy(
                rows_v[p], out_hbm.at[pl.ds(base + c * _CH, _CH)], sem_o[p])
        if n_ch >= 2:
            w[n_ch - 2].wait()
        w[n_ch - 1].wait()

    return gather_k


def kernel(x, W):
    b, t, d = x.shape
    k = max(1, math.ceil(0.5 * t))

    x2 = x.reshape(b * t, d)
    srow = _scores(x2, W.reshape(1, d)).reshape(b, t)
    indices, gidx = _topk(srow, k)

    gather_fn = _make_sc_gather(b * k, d)
    selected = gather_fn(x2, gidx.reshape(b * k))
    return selected.reshape(b, k, d), indices.reshape(b, k), srow
